# baseline (speedup 1.0000x reference)
import functools

import jax
import jax.numpy as jnp
from jax import lax
from jax.experimental import pallas as pl
from jax.experimental.pallas import tpu as pltpu
from jax.experimental.pallas import tpu_sc as plsc

_LANES = 16


def _emb_lookup(B, R, D):
    info = plsc.get_sparse_core_info()
    nw = info.num_cores * info.num_subcores
    assert B % (8 * nw) == 0 and D % _LANES == 0
    bpw = B // nw

    mesh = plsc.VectorSubcoreMesh(core_axis_name="c", subcore_axis_name="s")

    @functools.partial(
        pl.kernel,
        mesh=mesh,
        out_type=jax.ShapeDtypeStruct((B, D), jnp.float32),
        scratch_types=[
            pltpu.VMEM((bpw,), jnp.int32),
            pltpu.VMEM((bpw,), jnp.int32),
            pltpu.VMEM((bpw // 2, 4 * D), jnp.float32),
            pltpu.VMEM((bpw, D), jnp.float32),
            pltpu.SemaphoreType.DMA,
        ],
        compiler_params=pltpu.CompilerParams(
            use_tc_tiling_on_sc=True, needs_layout_passes=False),
    )
    def emb(idx_hbm, t2_hbm, out_hbm, idx_v, hi_v, buf, rows_v, sem):
        wid = lax.axis_index("s") * info.num_cores + lax.axis_index("c")
        base = wid * bpw
        pltpu.sync_copy(idx_hbm.at[pl.ds(base, bpw)], idx_v)

        def hi_body(k, _):
            v = idx_v[pl.ds(k * _LANES, _LANES)]
            hi_v[pl.ds(k * _LANES, _LANES)] = lax.shift_right_logical(v, 2)
            return _

        lax.fori_loop(0, bpw // _LANES, hi_body, 0, unroll=4)

        half = bpw // 2
        for c in range(2):
            pltpu.async_copy(
                t2_hbm.at[hi_v.at[pl.ds(c * half, half)]], buf, sem).wait()

            def sel_body(g, _, c=c):
                kvec = lax.iota(jnp.int32, _LANES) + g * _LANES
                off = c * half + g * _LANES
                lo = lax.bitwise_and(idx_v[pl.ds(off, _LANES)], 3)
                col0 = lo * D
                kabs = kvec + c * half

                def d_body(d, _):
                    for u in range(4):
                        dv = jnp.full((_LANES,), d * 4 + u, jnp.int32)
                        val = plsc.load_gather(buf, [kvec, col0 + dv])
                        plsc.store_scatter(rows_v, [kabs, dv], val)
                    return _

                lax.fori_loop(0, D // 4, d_body, 0)
                return _

            lax.fori_loop(0, half // _LANES, sel_body, 0)
        pltpu.sync_copy(rows_v, out_hbm.at[pl.ds(base, bpw)])

    return emb


def kernel(sentences, embed_table):
    (B,) = sentences.shape
    V, D = embed_table.shape
    t2 = embed_table.reshape(V // 4, 4 * D)
    return _emb_lookup(B, V // 4, D)(sentences.astype(jnp.int32), t2)


# speedup vs baseline: 2.8524x; 2.8524x over previous
import functools

import jax
import jax.numpy as jnp
from jax import lax
from jax.experimental import pallas as pl
from jax.experimental.pallas import tpu as pltpu
from jax.experimental.pallas import tpu_sc as plsc

_LANES = 16


def _emb_lookup(B, V, D):
    info = plsc.get_sparse_core_info()
    nw = info.num_cores * info.num_subcores
    assert B % (8 * nw) == 0 and D % _LANES == 0 and V % 8 == 0
    bpw = B // nw

    mesh = plsc.VectorSubcoreMesh(core_axis_name="c", subcore_axis_name="s")

    @functools.partial(
        pl.kernel,
        mesh=mesh,
        out_type=jax.ShapeDtypeStruct((B, D), jnp.float32),
        scratch_types=[
            pltpu.VMEM((bpw,), jnp.int32),
            pltpu.VMEM((bpw, D), jnp.float32),
            pltpu.SemaphoreType.DMA,
        ],
        compiler_params=pltpu.CompilerParams(use_tc_tiling_on_sc=True),
    )
    def emb(idx_hbm, t3_hbm, out_hbm, idx_v, rows_v, sem):
        wid = lax.axis_index("s") * info.num_cores + lax.axis_index("c")
        base = wid * bpw
        pltpu.sync_copy(idx_hbm.at[pl.ds(base, bpw)], idx_v)

        def g_body(g, _):
            v = idx_v[pl.ds(g * _LANES, _LANES)]
            for j in range(_LANES):
                s = v[j]
                hi = lax.shift_right_logical(s, 3)
                lo = lax.bitwise_and(s, 7)
                pltpu.async_copy(
                    t3_hbm.at[hi, lo], rows_v.at[g * _LANES + j], sem)
            return _

        lax.fori_loop(0, bpw // _LANES, g_body, 0)
        pltpu.make_async_copy(out_hbm.at[pl.ds(base, bpw)], rows_v, sem).wait()
        pltpu.sync_copy(rows_v, out_hbm.at[pl.ds(base, bpw)])

    return emb


def kernel(sentences, embed_table):
    (B,) = sentences.shape
    V, D = embed_table.shape
    t3 = embed_table.reshape(V // 8, 8, D)
    return _emb_lookup(B, V, D)(sentences.astype(jnp.int32), t3)
